# Initial kernel scaffold; baseline (speedup 1.0000x reference)
import functools

import jax
import jax.numpy as jnp
from jax import lax
from jax.experimental import pallas as pl
from jax.experimental.pallas import tpu as pltpu
from jax.experimental.pallas import tpu_sc as plsc

N_NODES = 10000
N_HIDDEN = 128
N_EVENTS = 320000
N_TYPES = 2

_LANES = 16
_NC = 2
_NS = 16
_NW = _NC * _NS
_CHUNK = N_EVENTS // _NW


def _table_body(emb_ref, w_ref, b_ref, out_ref):
    w = w_ref[...]
    s = 0.5 * (w[:, :N_HIDDEN] + w[:, N_HIDDEN:])
    t = lax.dot_general(emb_ref[...], s, (((1,), (1,)), ((), ())),
                        preferred_element_type=jnp.float32)
    out_ref[...] = t + 0.5 * b_ref[...]


def _make_table(embeddings, omega_W, omega_b):
    return pl.pallas_call(
        _table_body,
        out_shape=jax.ShapeDtypeStruct((N_NODES, N_TYPES), jnp.float32),
    )(embeddings, omega_W, omega_b.reshape(1, N_TYPES))


def _sc_body(u_hbm, v_hbm, k_hbm, tab_hbm, g_hbm, u_v, v_v, k_v, tab_v, g_v):
    wid = lax.axis_index("s") * _NC + lax.axis_index("c")
    base = wid * _CHUNK
    pltpu.sync_copy(u_hbm.at[pl.ds(base, _CHUNK)], u_v)
    pltpu.sync_copy(v_hbm.at[pl.ds(base, _CHUNK)], v_v)
    pltpu.sync_copy(k_hbm.at[pl.ds(base, _CHUNK)], k_v)
    pltpu.sync_copy(tab_hbm, tab_v)

    def body(i, carry):
        sl = pl.ds(i * _LANES, _LANES)
        uu = u_v[sl]
        vv = v_v[sl]
        kk = k_v[sl]
        gu = plsc.load_gather(tab_v, [uu, kk])
        gv = plsc.load_gather(tab_v, [vv, kk])
        g_v[sl] = gu + gv
        return carry

    lax.fori_loop(0, _CHUNK // _LANES, body, 0)
    pltpu.sync_copy(g_v, g_hbm.at[pl.ds(base, _CHUNK)])


def _gather_g(u, v, event_type, table):
    mesh = plsc.VectorSubcoreMesh(core_axis_name="c", subcore_axis_name="s")
    kern = functools.partial(
        pl.kernel,
        mesh=mesh,
        out_type=jax.ShapeDtypeStruct((N_EVENTS,), jnp.float32),
        scratch_types=[
            pltpu.VMEM((_CHUNK,), jnp.int32),
            pltpu.VMEM((_CHUNK,), jnp.int32),
            pltpu.VMEM((_CHUNK,), jnp.int32),
            pltpu.VMEM((N_NODES, N_TYPES), jnp.float32),
            pltpu.VMEM((_CHUNK,), jnp.float32),
        ],
    )(_sc_body)
    return kern(u, v, event_type, table)


def _softplus_body(psi_ref, g_ref, k_ref, out_ref):
    psi0 = psi_ref[0]
    psi1 = psi_ref[1]
    psi_e = jnp.where(k_ref[...] == 0, psi0, psi1)
    r = jnp.clip(g_ref[...] / psi_e, -75.0, 75.0)
    out_ref[...] = psi_e * jnp.log(1.0 + jnp.exp(r))


def _softplus(g2d, k2d, psi):
    return pl.pallas_call(
        _softplus_body,
        out_shape=jax.ShapeDtypeStruct(g2d.shape, jnp.float32),
        in_specs=[
            pl.BlockSpec(memory_space=pltpu.SMEM),
            pl.BlockSpec(g2d.shape, lambda: (0, 0)),
            pl.BlockSpec(g2d.shape, lambda: (0, 0)),
        ],
    )(psi, g2d, k2d)


def kernel(embeddings, omega_W, omega_b, psi, u, v, event_type):
    u = u.astype(jnp.int32)
    v = v.astype(jnp.int32)
    event_type = event_type.astype(jnp.int32)

    table = _make_table(embeddings, omega_W, omega_b)
    g = _gather_g(u, v, event_type, table)
    rows = N_EVENTS // 128
    out = _softplus(g.reshape(rows, 128), event_type.reshape(rows, 128), psi)
    return out.reshape(-1)


# speedup vs baseline: 44.3169x; 44.3169x over previous
import functools

import jax
import jax.numpy as jnp
from jax import lax
from jax.experimental import pallas as pl
from jax.experimental.pallas import tpu as pltpu
from jax.experimental.pallas import tpu_sc as plsc

N_NODES = 10000
N_HIDDEN = 128
N_EVENTS = 320000
N_TYPES = 2

_LANES = 16
_NC = 2
_NS = 16
_NW = _NC * _NS
_CHUNK = N_EVENTS // _NW


def _table_body(emb_ref, w_ref, b_ref, out_ref):
    w = w_ref[...]
    s = 0.5 * (w[:, :N_HIDDEN] + w[:, N_HIDDEN:])
    t = lax.dot_general(emb_ref[...], s, (((1,), (1,)), ((), ())),
                        preferred_element_type=jnp.float32)
    out_ref[...] = t + 0.5 * b_ref[...]


def _make_table(embeddings, omega_W, omega_b):
    return pl.pallas_call(
        _table_body,
        out_shape=jax.ShapeDtypeStruct((N_NODES, N_TYPES), jnp.float32),
    )(embeddings, omega_W, omega_b.reshape(1, N_TYPES))


def _sc_body(u_hbm, v_hbm, k_hbm, tab_hbm, g_hbm, u_v, v_v, k_v, tab_v, g_v):
    wid = lax.axis_index("s") * _NC + lax.axis_index("c")
    base = wid * _CHUNK
    pltpu.sync_copy(u_hbm.at[pl.ds(base, _CHUNK)], u_v)
    pltpu.sync_copy(v_hbm.at[pl.ds(base, _CHUNK)], v_v)
    pltpu.sync_copy(k_hbm.at[pl.ds(base, _CHUNK)], k_v)
    pltpu.sync_copy(tab_hbm, tab_v)

    def body(i, carry):
        sl = pl.ds(i * _LANES, _LANES)
        kk = k_v[sl]
        iu = u_v[sl] * N_TYPES + kk
        iv = v_v[sl] * N_TYPES + kk
        gu = plsc.load_gather(tab_v, [iu])
        gv = plsc.load_gather(tab_v, [iv])
        g_v[sl] = gu + gv
        return carry

    lax.fori_loop(0, _CHUNK // _LANES, body, 0)
    pltpu.sync_copy(g_v, g_hbm.at[pl.ds(base, _CHUNK)])


def _gather_g(u, v, event_type, table):
    mesh = plsc.VectorSubcoreMesh(core_axis_name="c", subcore_axis_name="s")
    kern = functools.partial(
        pl.kernel,
        mesh=mesh,
        out_type=jax.ShapeDtypeStruct((N_EVENTS,), jnp.float32),
        scratch_types=[
            pltpu.VMEM((_CHUNK,), jnp.int32),
            pltpu.VMEM((_CHUNK,), jnp.int32),
            pltpu.VMEM((_CHUNK,), jnp.int32),
            pltpu.VMEM((N_NODES * N_TYPES,), jnp.float32),
            pltpu.VMEM((_CHUNK,), jnp.float32),
        ],
        compiler_params=pltpu.CompilerParams(needs_layout_passes=False),
    )(_sc_body)
    return kern(u, v, event_type, table)


def _softplus_body(psi_ref, g_ref, k_ref, out_ref):
    psi0 = psi_ref[0]
    psi1 = psi_ref[1]
    psi_e = jnp.where(k_ref[...] == 0, psi0, psi1)
    r = jnp.clip(g_ref[...] / psi_e, -75.0, 75.0)
    out_ref[...] = psi_e * jnp.log(1.0 + jnp.exp(r))


def _softplus(g2d, k2d, psi):
    return pl.pallas_call(
        _softplus_body,
        out_shape=jax.ShapeDtypeStruct(g2d.shape, jnp.float32),
        in_specs=[
            pl.BlockSpec(memory_space=pltpu.SMEM),
            pl.BlockSpec(g2d.shape, lambda: (0, 0)),
            pl.BlockSpec(g2d.shape, lambda: (0, 0)),
        ],
    )(psi, g2d, k2d)


def kernel(embeddings, omega_W, omega_b, psi, u, v, event_type):
    u = u.astype(jnp.int32)
    v = v.astype(jnp.int32)
    event_type = event_type.astype(jnp.int32)

    table = _make_table(embeddings, omega_W, omega_b)
    g = _gather_g(u, v, event_type, table.reshape(-1))
    rows = N_EVENTS // 128
    out = _softplus(g.reshape(rows, 128), event_type.reshape(rows, 128), psi)
    return out.reshape(-1)


# speedup vs baseline: 48.4066x; 1.0923x over previous
import functools

import jax
import jax.numpy as jnp
from jax import lax
from jax.experimental import pallas as pl
from jax.experimental.pallas import tpu as pltpu
from jax.experimental.pallas import tpu_sc as plsc

N_NODES = 10000
N_HIDDEN = 128
N_EVENTS = 320000
N_TYPES = 2

_LANES = 16
_NC = 2
_NS = 16
_NW = _NC * _NS
_CHUNK = N_EVENTS // _NW


def _table_body(emb_ref, w_ref, b_ref, out_ref):
    w = w_ref[...]
    s = 0.5 * (w[:, :N_HIDDEN] + w[:, N_HIDDEN:])
    t = lax.dot_general(emb_ref[...], s, (((1,), (1,)), ((), ())),
                        preferred_element_type=jnp.float32)
    out_ref[...] = t + 0.5 * b_ref[...]


def _make_table(embeddings, omega_W, omega_b):
    return pl.pallas_call(
        _table_body,
        out_shape=jax.ShapeDtypeStruct((N_NODES, N_TYPES), jnp.float32),
    )(embeddings, omega_W, omega_b.reshape(1, N_TYPES))


def _sc_body(u_hbm, v_hbm, k_hbm, tab_hbm, g_hbm, u_v, v_v, k_v, tab_v, g_v,
             sem):
    wid = lax.axis_index("s") * _NC + lax.axis_index("c")
    base = wid * _CHUNK
    cu = pltpu.async_copy(u_hbm.at[pl.ds(base, _CHUNK)], u_v, sem)
    cv = pltpu.async_copy(v_hbm.at[pl.ds(base, _CHUNK)], v_v, sem)
    ck = pltpu.async_copy(k_hbm.at[pl.ds(base, _CHUNK)], k_v, sem)
    ct = pltpu.async_copy(tab_hbm, tab_v, sem)
    cu.wait()
    cv.wait()
    ck.wait()
    ct.wait()

    @plsc.parallel_loop(0, _CHUNK // _LANES, unroll=8)
    def _(i):
        sl = pl.ds(i * _LANES, _LANES)
        kk = k_v[sl]
        iu = u_v[sl] * N_TYPES + kk
        iv = v_v[sl] * N_TYPES + kk
        g_v[sl] = plsc.load_gather(tab_v, [iu]) + plsc.load_gather(tab_v, [iv])

    pltpu.sync_copy(g_v, g_hbm.at[pl.ds(base, _CHUNK)])


def _gather_g(u, v, event_type, table):
    mesh = plsc.VectorSubcoreMesh(core_axis_name="c", subcore_axis_name="s")
    kern = functools.partial(
        pl.kernel,
        mesh=mesh,
        out_type=jax.ShapeDtypeStruct((N_EVENTS,), jnp.float32),
        scratch_types=[
            pltpu.VMEM((_CHUNK,), jnp.int32),
            pltpu.VMEM((_CHUNK,), jnp.int32),
            pltpu.VMEM((_CHUNK,), jnp.int32),
            pltpu.VMEM((N_NODES * N_TYPES,), jnp.float32),
            pltpu.VMEM((_CHUNK,), jnp.float32),
            pltpu.SemaphoreType.DMA,
        ],
        compiler_params=pltpu.CompilerParams(needs_layout_passes=False),
    )(_sc_body)
    return kern(u, v, event_type, table)


def _softplus_body(psi_ref, g_ref, k_ref, out_ref):
    psi0 = psi_ref[0]
    psi1 = psi_ref[1]
    psi_e = jnp.where(k_ref[...] == 0, psi0, psi1)
    r = jnp.clip(g_ref[...] / psi_e, -75.0, 75.0)
    out_ref[...] = psi_e * jnp.log(1.0 + jnp.exp(r))


def _softplus(g2d, k2d, psi):
    return pl.pallas_call(
        _softplus_body,
        out_shape=jax.ShapeDtypeStruct(g2d.shape, jnp.float32),
        in_specs=[
            pl.BlockSpec(memory_space=pltpu.SMEM),
            pl.BlockSpec(g2d.shape, lambda: (0, 0)),
            pl.BlockSpec(g2d.shape, lambda: (0, 0)),
        ],
    )(psi, g2d, k2d)


def kernel(embeddings, omega_W, omega_b, psi, u, v, event_type):
    u = u.astype(jnp.int32)
    v = v.astype(jnp.int32)
    event_type = event_type.astype(jnp.int32)

    table = _make_table(embeddings, omega_W, omega_b)
    g = _gather_g(u, v, event_type, table.reshape(-1))
    rows = N_EVENTS // 128
    out = _softplus(g.reshape(rows, 128), event_type.reshape(rows, 128), psi)
    return out.reshape(-1)


# speedup vs baseline: 50.1022x; 1.0350x over previous
import functools

import jax
import jax.numpy as jnp
from jax import lax
from jax.experimental import pallas as pl
from jax.experimental.pallas import tpu as pltpu
from jax.experimental.pallas import tpu_sc as plsc

N_NODES = 10000
N_HIDDEN = 128
N_EVENTS = 320000
N_TYPES = 2

_LANES = 16
_NC = 2
_NS = 16
_NW = _NC * _NS
_CHUNK = N_EVENTS // _NW


def _table_body(emb_ref, w_ref, b_ref, out_ref):
    w = w_ref[...]
    s = 0.5 * (w[:, :N_HIDDEN] + w[:, N_HIDDEN:])
    t = lax.dot_general(emb_ref[...], s, (((1,), (1,)), ((), ())),
                        preferred_element_type=jnp.float32)
    out_ref[...] = t + 0.5 * b_ref[...]


def _make_table(embeddings, omega_W, omega_b):
    return pl.pallas_call(
        _table_body,
        out_shape=jax.ShapeDtypeStruct((N_NODES, N_TYPES), jnp.float32),
    )(embeddings, omega_W, omega_b.reshape(1, N_TYPES))


_LOG_SCALE = 0.6931471805599453 / 8388608.0
_LOG_BIAS = 0.6931471805599453 * 127.0430357


def _sc_body(u_hbm, v_hbm, k_hbm, tab_hbm, psi_hbm, out_hbm,
             u_v, v_v, k_v, tab_v, psi_v, o_v, sem):
    wid = lax.axis_index("s") * _NC + lax.axis_index("c")
    base = wid * _CHUNK
    cu = pltpu.async_copy(u_hbm.at[pl.ds(base, _CHUNK)], u_v, sem)
    cv = pltpu.async_copy(v_hbm.at[pl.ds(base, _CHUNK)], v_v, sem)
    ck = pltpu.async_copy(k_hbm.at[pl.ds(base, _CHUNK)], k_v, sem)
    ct = pltpu.async_copy(tab_hbm, tab_v, sem)
    cp = pltpu.async_copy(psi_hbm, psi_v, sem)
    cu.wait()
    cv.wait()
    ck.wait()
    ct.wait()
    cp.wait()

    @plsc.parallel_loop(0, _CHUNK // _LANES, unroll=8)
    def _(i):
        sl = pl.ds(i * _LANES, _LANES)
        kk = k_v[sl]
        iu = u_v[sl] * N_TYPES + kk
        iv = v_v[sl] * N_TYPES + kk
        g = plsc.load_gather(tab_v, [iu]) + plsc.load_gather(tab_v, [iv])
        psi_e = plsc.load_gather(psi_v, [kk])
        r = jnp.clip(g / psi_e, -75.0, 75.0)
        x = 1.0 + jnp.exp(r)
        y = plsc.bitcast(x, jnp.int32).astype(jnp.float32) * _LOG_SCALE - _LOG_BIAS
        y = y - 1.0 + x * jnp.exp(-y)
        y = y - 1.0 + x * jnp.exp(-y)
        o_v[sl] = psi_e * y

    pltpu.sync_copy(o_v, out_hbm.at[pl.ds(base, _CHUNK)])


def _sc_intensity(u, v, event_type, table, psi):
    mesh = plsc.VectorSubcoreMesh(core_axis_name="c", subcore_axis_name="s")
    kern = functools.partial(
        pl.kernel,
        mesh=mesh,
        out_type=jax.ShapeDtypeStruct((N_EVENTS,), jnp.float32),
        scratch_types=[
            pltpu.VMEM((_CHUNK,), jnp.int32),
            pltpu.VMEM((_CHUNK,), jnp.int32),
            pltpu.VMEM((_CHUNK,), jnp.int32),
            pltpu.VMEM((N_NODES * N_TYPES,), jnp.float32),
            pltpu.VMEM((N_TYPES,), jnp.float32),
            pltpu.VMEM((_CHUNK,), jnp.float32),
            pltpu.SemaphoreType.DMA,
        ],
        compiler_params=pltpu.CompilerParams(needs_layout_passes=False),
    )(_sc_body)
    return kern(u, v, event_type, table, psi)


def kernel(embeddings, omega_W, omega_b, psi, u, v, event_type):
    u = u.astype(jnp.int32)
    v = v.astype(jnp.int32)
    event_type = event_type.astype(jnp.int32)

    table = _make_table(embeddings, omega_W, omega_b)
    return _sc_intensity(u, v, event_type, table.reshape(-1), psi)


# speedup vs baseline: 66.6345x; 1.3300x over previous
import functools

import jax
import jax.numpy as jnp
from jax import lax
from jax.experimental import pallas as pl
from jax.experimental.pallas import tpu as pltpu
from jax.experimental.pallas import tpu_sc as plsc

N_NODES = 10000
N_HIDDEN = 128
N_EVENTS = 320000
N_TYPES = 2

_LANES = 16
_NC = 2
_NS = 16
_NW = _NC * _NS
_CHUNK = N_EVENTS // _NW


def _table_body(emb_ref, w_ref, b_ref, psi_ref, out_ref):
    w = w_ref[...]
    s = 0.5 * (w[:, :N_HIDDEN] + w[:, N_HIDDEN:])
    t = lax.dot_general(s, emb_ref[...], (((1,), (1,)), ((), ())),
                        preferred_element_type=jnp.float32)
    out_ref[pl.ds(0, N_NODES)] = (t[0, :] + 0.5 * b_ref[0]) / psi_ref[0]
    out_ref[pl.ds(N_NODES, N_NODES)] = (t[1, :] + 0.5 * b_ref[1]) / psi_ref[1]


def _make_table(embeddings, omega_W, omega_b, psi):
    return pl.pallas_call(
        _table_body,
        out_shape=jax.ShapeDtypeStruct((N_TYPES * N_NODES,), jnp.float32),
        in_specs=[
            pl.BlockSpec((N_NODES, N_HIDDEN), lambda: (0, 0)),
            pl.BlockSpec((N_TYPES, 2 * N_HIDDEN), lambda: (0, 0)),
            pl.BlockSpec(memory_space=pltpu.SMEM),
            pl.BlockSpec(memory_space=pltpu.SMEM),
        ],
    )(embeddings, omega_W, omega_b, psi)


_LOG_SCALE = 0.6931471805599453 / 8388608.0
_LOG_BIAS = 0.6931471805599453 * 126.9569643


_SEG0 = 4992
_SEG1 = _CHUNK - _SEG0


def _sc_body(u_hbm, v_hbm, k_hbm, tab_hbm, psi_hbm, out_hbm,
             u_v, v_v, k_v, tab_v, psi_v, o_v, sem, semw):
    wid = lax.axis_index("s") * _NC + lax.axis_index("c")
    base = wid * _CHUNK
    ct = pltpu.async_copy(tab_hbm, tab_v, sem)
    cp = pltpu.async_copy(psi_hbm, psi_v, sem)
    seg0 = [pltpu.async_copy(h.at[pl.ds(base, _SEG0)], d.at[pl.ds(0, _SEG0)], sem)
            for h, d in ((u_hbm, u_v), (v_hbm, v_v), (k_hbm, k_v))]
    seg1 = [pltpu.async_copy(h.at[pl.ds(base + _SEG0, _SEG1)],
                             d.at[pl.ds(_SEG0, _SEG1)], sem)
            for h, d in ((u_hbm, u_v), (v_hbm, v_v), (k_hbm, k_v))]
    ct.wait()
    cp.wait()
    for c in seg0:
        c.wait()

    def softplus_block(lo, n):
        @plsc.parallel_loop(lo // _LANES, (lo + n) // _LANES, unroll=4)
        def _(i):
            sl = pl.ds(i * _LANES, _LANES)
            kb = k_v[sl] * N_NODES
            r = (plsc.load_gather(tab_v, [kb + u_v[sl]])
                 + plsc.load_gather(tab_v, [kb + v_v[sl]]))
            psi_e = plsc.load_gather(psi_v, [k_v[sl]])
            r = jnp.clip(r, -75.0, 75.0)
            x = 1.0 + jnp.exp(r)
            y = (plsc.bitcast(x, jnp.int32).astype(jnp.float32) * _LOG_SCALE
                 - _LOG_BIAS)
            y = y - 1.0 + x * jnp.exp(-y)
            o_v[sl] = psi_e * y

    softplus_block(0, _SEG0)
    w0 = pltpu.async_copy(o_v.at[pl.ds(0, _SEG0)],
                          out_hbm.at[pl.ds(base, _SEG0)], semw)
    for c in seg1:
        c.wait()
    softplus_block(_SEG0, _SEG1)
    pltpu.sync_copy(o_v.at[pl.ds(_SEG0, _SEG1)],
                    out_hbm.at[pl.ds(base + _SEG0, _SEG1)])
    w0.wait()


def _sc_intensity(u, v, event_type, table, psi):
    mesh = plsc.VectorSubcoreMesh(core_axis_name="c", subcore_axis_name="s")
    kern = functools.partial(
        pl.kernel,
        mesh=mesh,
        out_type=jax.ShapeDtypeStruct((N_EVENTS,), jnp.float32),
        scratch_types=[
            pltpu.VMEM((_CHUNK,), jnp.int32),
            pltpu.VMEM((_CHUNK,), jnp.int32),
            pltpu.VMEM((_CHUNK,), jnp.int32),
            pltpu.VMEM((N_TYPES * N_NODES,), jnp.float32),
            pltpu.VMEM((N_TYPES,), jnp.float32),
            pltpu.VMEM((_CHUNK,), jnp.float32),
            pltpu.SemaphoreType.DMA,
            pltpu.SemaphoreType.DMA,
        ],
        compiler_params=pltpu.CompilerParams(needs_layout_passes=False),
    )(_sc_body)
    return kern(u, v, event_type, table, psi)


def kernel(embeddings, omega_W, omega_b, psi, u, v, event_type):
    u = u.astype(jnp.int32)
    v = v.astype(jnp.int32)
    event_type = event_type.astype(jnp.int32)

    table = _make_table(embeddings, omega_W, omega_b, psi)
    return _sc_intensity(u, v, event_type, table, psi)
